# Initial kernel scaffold; baseline (speedup 1.0000x reference)
import jax
import jax.numpy as jnp
from jax.experimental import pallas as pl


def kernel(c1_w, c1_b, c2_w, c2_b, c3_w, c3_b, head_w, head_b, x_nchw):
    raise NotImplementedError("write your pallas kernel here")


# speedup vs baseline: 2.1615x; 2.1615x over previous
import jax
import jax.numpy as jnp
from jax.experimental import pallas as pl
from jax.experimental.pallas import tpu as pltpu

B = 8
KS = 5


def _net_kernel(x_ref, w1_ref, b1_ref, w2_ref, b2_ref, w3_ref, b3_ref,
                wh_ref, bh_ref, o_ref):
    f32 = jnp.float32

    def pool2(u):
        b, h, w, c = u.shape
        u = u.reshape(b, h // 2, 2, w // 2, 2, c)
        return jnp.max(u, axis=(2, 4))

    x = x_ref[...]
    p15 = jnp.concatenate(
        [x[:, :, dj:dj + 32, :] for dj in range(KS)], axis=-1)
    p75 = jnp.concatenate(
        [p15[:, di:di + 32] for di in range(KS)], axis=-1)
    u1 = jnp.dot(p75.reshape(B * 32 * 32, 100), w1_ref[...],
                 preferred_element_type=f32).reshape(B, 32, 32, 32)
    y1 = pool2(u1) + b1_ref[...].reshape(1, 1, 1, 32)
    y1 = y1.astype(jnp.bfloat16)

    y1p = jnp.pad(y1, ((0, 0), (2, 2), (2, 2), (0, 0)))
    p2 = jnp.concatenate(
        [y1p[:, :, dj:dj + 16, :] for dj in range(KS)], axis=-1)
    u2 = jnp.dot(p2.reshape(B * 20 * 16, 160), w2_ref[...],
                 preferred_element_type=f32).reshape(B, 20, 16, 160)
    c2 = sum(u2[:, di:di + 16, :, di * 32:(di + 1) * 32] for di in range(KS))
    y2 = pool2(c2) + b2_ref[...].reshape(1, 1, 1, 32)
    y2 = y2.astype(jnp.bfloat16)

    y2p = jnp.pad(y2, ((0, 0), (2, 2), (2, 2), (0, 0)))
    p3 = jnp.concatenate(
        [y2p[:, :, dj:dj + 8, :] for dj in range(KS)], axis=-1)
    u3 = jnp.dot(p3.reshape(B * 12 * 8, 160), w3_ref[...],
                 preferred_element_type=f32).reshape(B, 12, 8, 320)
    c3 = sum(u3[:, di:di + 8, :, di * 64:(di + 1) * 64] for di in range(KS))
    y3 = pool2(c3) + b3_ref[...].reshape(1, 1, 1, 64)
    y3 = y3.astype(jnp.bfloat16)

    acc = bh_ref[...].astype(f32)
    out = jnp.zeros((B, 128), f32) + acc
    for i in range(16):
        h, w = divmod(i, 4)
        out = out + jnp.dot(y3[:, h, w, :], wh_ref[i * 64:(i + 1) * 64, :],
                            preferred_element_type=f32)
    o_ref[...] = out


def kernel(c1_w, c1_b, c2_w, c2_b, c3_w, c3_b, head_w, head_b, x_nchw):
    N = x_nchw.shape[0]
    bf16 = jnp.bfloat16

    w1 = jnp.pad(c1_w, ((0, 0), (0, 1), (0, 0))).reshape(100, 128)[:, :32]
    w2 = c2_w[:, :32, :32].reshape(KS, KS, 32, 32)
    w2 = jnp.transpose(w2, (1, 2, 0, 3)).reshape(160, 160)
    w3 = c3_w[:, :32, :64].reshape(KS, KS, 32, 64)
    w3 = jnp.transpose(w3, (1, 2, 0, 3)).reshape(160, 320)
    wh = head_w.reshape(4, 4, 128, 128)[:, :, :64, :].reshape(1024, 128)

    b1 = c1_b[:, :32]
    b2 = c2_b[:, :32]
    b3 = c3_b[:, :64]
    bh = head_b.reshape(1, 128)

    x = jnp.transpose(x_nchw, (0, 2, 3, 1))
    x = jnp.pad(x, ((0, 0), (2, 2), (2, 2), (0, 1)))
    x = x.astype(bf16)

    out = pl.pallas_call(
        _net_kernel,
        out_shape=jax.ShapeDtypeStruct((N, 128), jnp.float32),
        grid=(N // B,),
        in_specs=[
            pl.BlockSpec((B, 36, 36, 4), lambda n: (n, 0, 0, 0)),
            pl.BlockSpec((100, 32), lambda n: (0, 0)),
            pl.BlockSpec((1, 32), lambda n: (0, 0)),
            pl.BlockSpec((160, 160), lambda n: (0, 0)),
            pl.BlockSpec((1, 32), lambda n: (0, 0)),
            pl.BlockSpec((160, 320), lambda n: (0, 0)),
            pl.BlockSpec((1, 64), lambda n: (0, 0)),
            pl.BlockSpec((1024, 128), lambda n: (0, 0)),
            pl.BlockSpec((1, 128), lambda n: (0, 0)),
        ],
        out_specs=pl.BlockSpec((B, 128), lambda n: (n, 0)),
        compiler_params=pltpu.CompilerParams(
            dimension_semantics=("parallel",)),
    )(x, w1.astype(bf16), b1, w2.astype(bf16), b2, w3.astype(bf16), b3,
      wh.astype(bf16), bh)
    return out[:, :10]


# speedup vs baseline: 6.6006x; 3.0537x over previous
import jax
import jax.numpy as jnp
from jax.experimental import pallas as pl
from jax.experimental.pallas import tpu as pltpu

B = 8
KS = 5


def _rotmax(t, k):
    r = jnp.concatenate([t[..., k:], t[..., :k]], axis=-1)
    return jnp.maximum(t, r)


def _net_kernel(x_ref, t1_ref, b1_ref, t2_ref, b2_ref, t3_ref, b3_ref,
                wh_ref, bh_ref, o_ref):
    f32 = jnp.float32
    x = x_ref[...]

    u1 = jnp.dot(x[:, 0:32, :].reshape(B * 32, 144), t1_ref[0:144, :],
                 preferred_element_type=f32)
    for di in range(1, KS):
        u1 = u1 + jnp.dot(x[:, di:di + 32, :].reshape(B * 32, 144),
                          t1_ref[di * 144:(di + 1) * 144, :],
                          preferred_element_type=f32)
    u1 = u1.reshape(B, 16, 2, 1024)
    m1 = _rotmax(jnp.max(u1, axis=2), 32) + b1_ref[...].reshape(1, 1024)
    y1 = jnp.pad(m1.astype(jnp.bfloat16), ((0, 0), (2, 2), (0, 0)))

    u2 = jnp.dot(y1[:, 0:16, :].reshape(B * 16, 1024), t2_ref[0:1024, :],
                 preferred_element_type=f32)
    for di in range(1, KS):
        u2 = u2 + jnp.dot(y1[:, di:di + 16, :].reshape(B * 16, 1024),
                          t2_ref[di * 1024:(di + 1) * 1024, :],
                          preferred_element_type=f32)
    u2 = u2.reshape(B, 8, 2, 512)
    m2 = _rotmax(jnp.max(u2, axis=2), 32) + b2_ref[...].reshape(1, 512)
    y2 = jnp.pad(m2.astype(jnp.bfloat16), ((0, 0), (2, 2), (0, 0)))

    u3 = jnp.dot(y2[:, 0:8, :].reshape(B * 8, 512), t3_ref[0:512, :],
                 preferred_element_type=f32)
    for di in range(1, KS):
        u3 = u3 + jnp.dot(y2[:, di:di + 8, :].reshape(B * 8, 512),
                          t3_ref[di * 512:(di + 1) * 512, :],
                          preferred_element_type=f32)
    u3 = u3.reshape(B, 4, 2, 512)
    m3 = _rotmax(jnp.max(u3, axis=2), 64) + b3_ref[...].reshape(1, 512)
    y3 = m3.astype(jnp.bfloat16)

    out = jnp.zeros((B, 128), f32) + bh_ref[...]
    for h in range(4):
        out = out + jnp.dot(y3[:, h, :], wh_ref[h * 512:(h + 1) * 512, :],
                            preferred_element_type=f32)
    o_ref[...] = out


def _toeplitz(w_dxio, e_dj_list, w_in, c_in, w_out, c_out):
    t = sum(jnp.einsum("xw,dio->dxiwo", e_dj_list[dj], w_dxio[:, dj])
            for dj in range(KS))
    return t.reshape(KS * w_in * c_in, w_out * c_out)


def kernel(c1_w, c1_b, c2_w, c2_b, c3_w, c3_b, head_w, head_b, x_nchw):
    N = x_nchw.shape[0]
    bf16 = jnp.bfloat16
    ar = jnp.arange

    w1 = jnp.pad(c1_w, ((0, 0), (0, 1), (0, 0)))[:, :, :32].reshape(KS, KS, 4, 32)
    e1 = [(ar(36)[:, None] == ar(32)[None, :] + dj).astype(jnp.float32)
          for dj in range(KS)]
    t1 = _toeplitz(w1, e1, 36, 4, 32, 32)

    w2 = c2_w[:, :32, :32].reshape(KS, KS, 32, 32)
    e2 = [(ar(32)[:, None] == 2 * (ar(16)[None, :] + dj - 2)).astype(jnp.float32)
          for dj in range(KS)]
    t2 = _toeplitz(w2, e2, 32, 32, 16, 32)

    w3 = c3_w[:, :32, :64].reshape(KS, KS, 32, 64)
    e3 = [(ar(16)[:, None] == 2 * (ar(8)[None, :] + dj - 2)).astype(jnp.float32)
          for dj in range(KS)]
    t3 = _toeplitz(w3, e3, 16, 32, 8, 64)

    wh4 = head_w.reshape(4, 4, 128, 128)[:, :, :64, :]
    eh = (ar(8)[:, None] == 2 * ar(4)[None, :]).astype(jnp.float32)
    wh = jnp.einsum("xw,hwcn->hxcn", eh, wh4).reshape(2048, 128)

    bias1 = jnp.tile(c1_b[:, :32], (1, 32))
    bias2 = jnp.tile(c2_b[:, :32], (1, 16))
    bias3 = jnp.tile(c3_b[:, :64], (1, 8))
    bh = head_b.reshape(1, 128)

    x = jnp.transpose(x_nchw, (0, 2, 3, 1))
    x = jnp.pad(x, ((0, 0), (2, 2), (2, 2), (0, 1)))
    x = x.reshape(N, 36, 144).astype(bf16)

    out = pl.pallas_call(
        _net_kernel,
        out_shape=jax.ShapeDtypeStruct((N, 128), jnp.float32),
        grid=(N // B,),
        in_specs=[
            pl.BlockSpec((B, 36, 144), lambda n: (n, 0, 0)),
            pl.BlockSpec((720, 1024), lambda n: (0, 0)),
            pl.BlockSpec((1, 1024), lambda n: (0, 0)),
            pl.BlockSpec((5120, 512), lambda n: (0, 0)),
            pl.BlockSpec((1, 512), lambda n: (0, 0)),
            pl.BlockSpec((2560, 512), lambda n: (0, 0)),
            pl.BlockSpec((1, 512), lambda n: (0, 0)),
            pl.BlockSpec((2048, 128), lambda n: (0, 0)),
            pl.BlockSpec((1, 128), lambda n: (0, 0)),
        ],
        out_specs=pl.BlockSpec((B, 128), lambda n: (n, 0)),
        compiler_params=pltpu.CompilerParams(
            dimension_semantics=("parallel",)),
    )(x, t1.astype(bf16), bias1, t2.astype(bf16), bias2,
      t3.astype(bf16), bias3, wh.astype(bf16), bh)
    return out[:, :10]


# speedup vs baseline: 17.3572x; 2.6296x over previous
import jax
import jax.numpy as jnp
from jax.experimental import pallas as pl
from jax.experimental.pallas import tpu as pltpu

B = 32
KS = 5


def _rotmax(t, k):
    r = jnp.concatenate([t[..., k:], t[..., :k]], axis=-1)
    return jnp.maximum(t, r)


def _conv(y, t_ref, h_out, k):
    u = jnp.dot(y[0:h_out].reshape(h_out * B, k), t_ref[0:k, :],
                preferred_element_type=jnp.float32)
    for di in range(1, KS):
        u = u + jnp.dot(y[di:di + h_out].reshape(h_out * B, k),
                        t_ref[di * k:(di + 1) * k, :],
                        preferred_element_type=jnp.float32)
    return u


def _pool(u, h2, n, rot, bias):
    m = jnp.max(u.reshape(h2, 2, B, n), axis=1)
    m = _rotmax(m, rot) + bias.reshape(1, 1, n)
    return m.astype(jnp.bfloat16)


def _net_kernel(x_ref, t1_ref, b1_ref, t2_ref, b2_ref, t3_ref, b3_ref,
                wh_ref, bh_ref, o_ref):
    pad_h = ((2, 2), (0, 0), (0, 0))

    u1 = _conv(x_ref[...], t1_ref, 32, 144)
    y1 = jnp.pad(_pool(u1, 16, 1024, 32, b1_ref[...]), pad_h)

    u2 = _conv(y1, t2_ref, 16, 1024)
    y2 = jnp.pad(_pool(u2, 8, 512, 32, b2_ref[...]), pad_h)

    u3 = _conv(y2, t3_ref, 8, 512)
    y3 = _pool(u3, 4, 512, 64, b3_ref[...])

    out = jnp.zeros((B, 128), jnp.float32) + bh_ref[...]
    for h in range(4):
        out = out + jnp.dot(y3[h], wh_ref[h * 512:(h + 1) * 512, :],
                            preferred_element_type=jnp.float32)
    o_ref[...] = out


def _toeplitz(w_dxio, e_dj_list, w_in, c_in, w_out, c_out):
    t = sum(jnp.einsum("xw,dio->dxiwo", e_dj_list[dj], w_dxio[:, dj])
            for dj in range(KS))
    return t.reshape(KS * w_in * c_in, w_out * c_out)


def kernel(c1_w, c1_b, c2_w, c2_b, c3_w, c3_b, head_w, head_b, x_nchw):
    N = x_nchw.shape[0]
    bf16 = jnp.bfloat16
    ar = jnp.arange

    w1 = jnp.pad(c1_w, ((0, 0), (0, 1), (0, 0)))[:, :, :32].reshape(KS, KS, 4, 32)
    e1 = [(ar(36)[:, None] == ar(32)[None, :] + dj).astype(jnp.float32)
          for dj in range(KS)]
    t1 = _toeplitz(w1, e1, 36, 4, 32, 32)

    w2 = c2_w[:, :32, :32].reshape(KS, KS, 32, 32)
    e2 = [(ar(32)[:, None] == 2 * (ar(16)[None, :] + dj - 2)).astype(jnp.float32)
          for dj in range(KS)]
    t2 = _toeplitz(w2, e2, 32, 32, 16, 32)

    w3 = c3_w[:, :32, :64].reshape(KS, KS, 32, 64)
    e3 = [(ar(16)[:, None] == 2 * (ar(8)[None, :] + dj - 2)).astype(jnp.float32)
          for dj in range(KS)]
    t3 = _toeplitz(w3, e3, 16, 32, 8, 64)

    wh4 = head_w.reshape(4, 4, 128, 128)[:, :, :64, :]
    eh = (ar(8)[:, None] == 2 * ar(4)[None, :]).astype(jnp.float32)
    wh = jnp.einsum("xw,hwcn->hxcn", eh, wh4).reshape(2048, 128)

    bias1 = jnp.tile(c1_b[:, :32], (1, 32))
    bias2 = jnp.tile(c2_b[:, :32], (1, 16))
    bias3 = jnp.tile(c3_b[:, :64], (1, 8))
    bh = head_b.reshape(1, 128)

    x = jnp.transpose(x_nchw, (2, 0, 3, 1))
    x = jnp.pad(x, ((2, 2), (0, 0), (2, 2), (0, 1)))
    x = x.reshape(36, N, 144).astype(bf16)

    out = pl.pallas_call(
        _net_kernel,
        out_shape=jax.ShapeDtypeStruct((N, 128), jnp.float32),
        grid=(N // B,),
        in_specs=[
            pl.BlockSpec((36, B, 144), lambda n: (0, n, 0)),
            pl.BlockSpec((720, 1024), lambda n: (0, 0)),
            pl.BlockSpec((1, 1024), lambda n: (0, 0)),
            pl.BlockSpec((5120, 512), lambda n: (0, 0)),
            pl.BlockSpec((1, 512), lambda n: (0, 0)),
            pl.BlockSpec((2560, 512), lambda n: (0, 0)),
            pl.BlockSpec((1, 512), lambda n: (0, 0)),
            pl.BlockSpec((2048, 128), lambda n: (0, 0)),
            pl.BlockSpec((1, 128), lambda n: (0, 0)),
        ],
        out_specs=pl.BlockSpec((B, 128), lambda n: (n, 0)),
        compiler_params=pltpu.CompilerParams(
            dimension_semantics=("parallel",)),
    )(x, t1.astype(bf16), bias1, t2.astype(bf16), bias2,
      t3.astype(bf16), bias3, wh.astype(bf16), bh)
    return out[:, :10]


# speedup vs baseline: 22.2586x; 1.2824x over previous
import jax
import jax.numpy as jnp
from jax.experimental import pallas as pl
from jax.experimental.pallas import tpu as pltpu

B = 32
KS = 5


def _conv(y, t_ref, h_out, k):
    u = jnp.dot(y[0:h_out].reshape(h_out * B, k), t_ref[0:k, :],
                preferred_element_type=jnp.float32)
    for di in range(1, KS):
        u = u + jnp.dot(y[di:di + h_out].reshape(h_out * B, k),
                        t_ref[di * k:(di + 1) * k, :],
                        preferred_element_type=jnp.float32)
    return u


def _pool(u, h2, n, bias):
    m = jnp.max(u.reshape(h2, 2, B, n), axis=1)
    m = jnp.maximum(m[..., :n // 2], m[..., n // 2:]) + bias.reshape(1, 1, n // 2)
    return m.astype(jnp.bfloat16)


def _net_kernel(x_ref, t1_ref, b1_ref, t2_ref, b2_ref, t3_ref, b3_ref,
                wh_ref, bh_ref, o_ref):
    pad_h = ((2, 2), (0, 0), (0, 0))

    u1 = _conv(x_ref[...], t1_ref, 32, 144)
    y1 = jnp.pad(_pool(u1, 16, 1024, b1_ref[...]), pad_h)

    u2 = _conv(y1, t2_ref, 16, 512)
    y2 = jnp.pad(_pool(u2, 8, 512, b2_ref[...]), pad_h)

    u3 = _conv(y2, t3_ref, 8, 256)
    y3 = _pool(u3, 4, 512, b3_ref[...])

    out = jnp.zeros((B, 128), jnp.float32) + bh_ref[...]
    for h in range(4):
        out = out + jnp.dot(y3[h], wh_ref[h * 256:(h + 1) * 256, :],
                            preferred_element_type=jnp.float32)
    o_ref[...] = out


def _toeplitz(w_dxio, e_dj_list, w_in, c_in, w_out, c_out):
    t = sum(jnp.einsum("xw,dio->dxiwo", e_dj_list[dj], w_dxio[:, dj])
            for dj in range(KS))
    k = KS * w_in * c_in
    t = t.reshape(k, w_out // 2, 2, c_out)
    t = jnp.transpose(t, (0, 2, 1, 3))
    return t.reshape(k, w_out * c_out)


def kernel(c1_w, c1_b, c2_w, c2_b, c3_w, c3_b, head_w, head_b, x_nchw):
    N = x_nchw.shape[0]
    bf16 = jnp.bfloat16
    ar = jnp.arange

    w1 = jnp.pad(c1_w, ((0, 0), (0, 1), (0, 0)))[:, :, :32].reshape(KS, KS, 4, 32)
    e1 = [(ar(36)[:, None] == ar(32)[None, :] + dj).astype(jnp.float32)
          for dj in range(KS)]
    t1 = _toeplitz(w1, e1, 36, 4, 32, 32)

    w2 = c2_w[:, :32, :32].reshape(KS, KS, 32, 32)
    e2 = [(ar(16)[:, None] == ar(16)[None, :] + dj - 2).astype(jnp.float32)
          for dj in range(KS)]
    t2 = _toeplitz(w2, e2, 16, 32, 16, 32)

    w3 = c3_w[:, :32, :64].reshape(KS, KS, 32, 64)
    e3 = [(ar(8)[:, None] == ar(8)[None, :] + dj - 2).astype(jnp.float32)
          for dj in range(KS)]
    t3 = _toeplitz(w3, e3, 8, 32, 8, 64)

    wh = head_w.reshape(4, 4, 128, 128)[:, :, :64, :].reshape(1024, 128)

    bias1 = jnp.tile(c1_b[:, :32], (1, 16))
    bias2 = jnp.tile(c2_b[:, :32], (1, 8))
    bias3 = jnp.tile(c3_b[:, :64], (1, 4))
    bh = head_b.reshape(1, 128)

    x = jnp.transpose(x_nchw, (2, 0, 3, 1))
    x = jnp.pad(x, ((2, 2), (0, 0), (2, 2), (0, 1)))
    x = x.reshape(36, N, 144).astype(bf16)

    out = pl.pallas_call(
        _net_kernel,
        out_shape=jax.ShapeDtypeStruct((N, 128), jnp.float32),
        grid=(N // B,),
        in_specs=[
            pl.BlockSpec((36, B, 144), lambda n: (0, n, 0)),
            pl.BlockSpec((720, 1024), lambda n: (0, 0)),
            pl.BlockSpec((1, 512), lambda n: (0, 0)),
            pl.BlockSpec((2560, 512), lambda n: (0, 0)),
            pl.BlockSpec((1, 256), lambda n: (0, 0)),
            pl.BlockSpec((1280, 512), lambda n: (0, 0)),
            pl.BlockSpec((1, 256), lambda n: (0, 0)),
            pl.BlockSpec((1024, 128), lambda n: (0, 0)),
            pl.BlockSpec((1, 128), lambda n: (0, 0)),
        ],
        out_specs=pl.BlockSpec((B, 128), lambda n: (n, 0)),
        compiler_params=pltpu.CompilerParams(
            dimension_semantics=("parallel",)),
    )(x, t1.astype(bf16), bias1, t2.astype(bf16), bias2,
      t3.astype(bf16), bias3, wh.astype(bf16), bh)
    return out[:, :10]


# speedup vs baseline: 24.8295x; 1.1155x over previous
import jax
import jax.numpy as jnp
from jax.experimental import pallas as pl
from jax.experimental.pallas import tpu as pltpu

B = 32
KS = 5


def _conv(y, t_ref, h_out, k):
    u = jnp.dot(y[0:h_out].reshape(h_out * B, k), t_ref[0:k, :],
                preferred_element_type=jnp.float32)
    for di in range(1, KS):
        u = u + jnp.dot(y[di:di + h_out].reshape(h_out * B, k),
                        t_ref[di * k:(di + 1) * k, :],
                        preferred_element_type=jnp.float32)
    return u


def _pool(u, h2, n, bias):
    m = jnp.max(u.reshape(h2, 2, B, n), axis=1)
    m = jnp.maximum(m[..., :n // 2], m[..., n // 2:]) + bias.reshape(1, 1, n // 2)
    return m.astype(jnp.bfloat16)


def _net_kernel(x2_ref, x1_ref, t1_ref, b1_ref, t2_ref, b2_ref, t3_ref,
                b3_ref, wh_ref, bh_ref, o_ref):
    f32 = jnp.float32
    pad_h = ((2, 2), (0, 0), (0, 0))

    u1 = (jnp.dot(x2_ref[0:32].reshape(32 * B, 216), t1_ref[0:216, :],
                  preferred_element_type=f32)
          + jnp.dot(x2_ref[2:34].reshape(32 * B, 216), t1_ref[216:432, :],
                    preferred_element_type=f32)
          + jnp.dot(x1_ref[...].reshape(32 * B, 108), t1_ref[432:540, :],
                    preferred_element_type=f32))
    y1 = jnp.pad(_pool(u1, 16, 1024, b1_ref[...]), pad_h)

    u2 = _conv(y1, t2_ref, 16, 512)
    y2 = jnp.pad(_pool(u2, 8, 512, b2_ref[...]), pad_h)

    u3 = _conv(y2, t3_ref, 8, 256)
    y3 = _pool(u3, 4, 512, b3_ref[...])

    out = jnp.zeros((B, 128), f32) + bh_ref[...]
    for h in range(4):
        out = out + jnp.dot(y3[h], wh_ref[h * 256:(h + 1) * 256, :],
                            preferred_element_type=f32)
    o_ref[...] = out


def _toeplitz(w_dxio, e_dj_list, w_in, c_in, w_out, c_out):
    t = sum(jnp.einsum("xw,dio->dxiwo", e_dj_list[dj], w_dxio[:, dj])
            for dj in range(KS))
    k = KS * w_in * c_in
    t = t.reshape(k, w_out // 2, 2, c_out)
    t = jnp.transpose(t, (0, 2, 1, 3))
    return t.reshape(k, w_out * c_out)


def kernel(c1_w, c1_b, c2_w, c2_b, c3_w, c3_b, head_w, head_b, x_nchw):
    N = x_nchw.shape[0]
    bf16 = jnp.bfloat16
    ar = jnp.arange

    w1 = c1_w[:, :, :32].reshape(KS, KS, 3, 32)
    e1 = [(ar(36)[:, None] == ar(32)[None, :] + dj).astype(jnp.float32)
          for dj in range(KS)]
    t1 = _toeplitz(w1, e1, 36, 3, 32, 32)

    w2 = c2_w[:, :32, :32].reshape(KS, KS, 32, 32)
    e2 = [(ar(16)[:, None] == ar(16)[None, :] + dj - 2).astype(jnp.float32)
          for dj in range(KS)]
    t2 = _toeplitz(w2, e2, 16, 32, 16, 32)

    w3 = c3_w[:, :32, :64].reshape(KS, KS, 32, 64)
    e3 = [(ar(8)[:, None] == ar(8)[None, :] + dj - 2).astype(jnp.float32)
          for dj in range(KS)]
    t3 = _toeplitz(w3, e3, 8, 32, 8, 64)

    wh = head_w.reshape(4, 4, 128, 128)[:, :, :64, :].reshape(1024, 128)

    bias1 = jnp.tile(c1_b[:, :32], (1, 16))
    bias2 = jnp.tile(c2_b[:, :32], (1, 8))
    bias3 = jnp.tile(c3_b[:, :64], (1, 4))
    bh = head_b.reshape(1, 128)

    x = jnp.transpose(x_nchw, (2, 0, 3, 1))
    x = jnp.pad(x, ((2, 2), (0, 0), (2, 2), (0, 0)))
    x = x.reshape(36, N, 108).astype(bf16)
    x2 = jnp.concatenate([x[0:34], x[1:35]], axis=-1)
    x1 = x[4:36]

    out = pl.pallas_call(
        _net_kernel,
        out_shape=jax.ShapeDtypeStruct((N, 128), jnp.float32),
        grid=(N // B,),
        in_specs=[
            pl.BlockSpec((34, B, 216), lambda n: (0, n, 0)),
            pl.BlockSpec((32, B, 108), lambda n: (0, n, 0)),
            pl.BlockSpec((540, 1024), lambda n: (0, 0)),
            pl.BlockSpec((1, 512), lambda n: (0, 0)),
            pl.BlockSpec((2560, 512), lambda n: (0, 0)),
            pl.BlockSpec((1, 256), lambda n: (0, 0)),
            pl.BlockSpec((1280, 512), lambda n: (0, 0)),
            pl.BlockSpec((1, 256), lambda n: (0, 0)),
            pl.BlockSpec((1024, 128), lambda n: (0, 0)),
            pl.BlockSpec((1, 128), lambda n: (0, 0)),
        ],
        out_specs=pl.BlockSpec((B, 128), lambda n: (n, 0)),
        compiler_params=pltpu.CompilerParams(
            dimension_semantics=("parallel",)),
    )(x2, x1, t1.astype(bf16), bias1, t2.astype(bf16), bias2,
      t3.astype(bf16), bias3, wh.astype(bf16), bh)
    return out[:, :10]
